# baseline (speedup 1.0000x reference)
import jax
import jax.numpy as jnp
import numpy as np
from jax.experimental import pallas as pl

_NUM = 1024
_FB = 8
_BB = 8

_PERM_NP = np.zeros((225, 225), np.float32)
for _m in range(75):
    for _c in range(3):
        _PERM_NP[75 * _c + _m, 3 * _m + _c] = 1.0

_SEL_NP = np.zeros((7, 192, 45), np.float32)
for _dx in range(7):
    for _c in range(15):
        for _ch in range(3):
            _SEL_NP[_dx, (4 * _c + _dx) * 3 + _ch, 3 * _c + _ch] = 1.0


def _agent_block_kernel(obs_ref, sel_ref, bd_ref, b165_ref, perm_ref,
                        wfe2t_ref, bfe2_ref, wc0t_ref, bc0_ref, wc1t_ref,
                        bc1_ref, wc2t_ref, bc2_ref, out_ref):
    f32 = jnp.float32
    x = obs_ref[...] * (1.0 / 255.0)

    y = jnp.zeros((_BB, 15, 165), f32)
    for dy in range(7):
        r_dy = x[:, dy // 4:dy // 4 + 15, dy % 4, :]
        for dx in range(7):
            c_k = jax.lax.dot_general(
                r_dy, sel_ref[dx], (((2,), (0,)), ((), ())),
                preferred_element_type=f32)
            y = y + jax.lax.dot_general(
                c_k, bd_ref[dy * 7 + dx], (((2,), (0,)), ((), ())),
                preferred_element_type=f32)
    y = y + b165_ref[0]

    def out_mat(lo, hi):
        cols = [y[:, :, o * 15:o * 15 + 15][:, :, :, None]
                for o in range(lo, hi)]
        return jnp.concatenate(cols, axis=3).reshape(_BB, 225, hi - lo)

    q = out_mat(0, 3)
    k = out_mat(3, 6)
    hpre = out_mat(6, 11)

    krows = []
    for a in range(3):
        ka = k[:, 75 * a:75 * a + 75, :]
        krows.append(jnp.concatenate(
            [ka[:, :, 0], ka[:, :, 1], ka[:, :, 2]], axis=1)[:, None, :])
    ktil = jnp.concatenate(krows, axis=1)
    s = jax.lax.dot_general(
        q, ktil, (((2,), (1,)), ((0,), (0,))),
        preferred_element_type=f32) * (1.0 / np.sqrt(147.0))

    m = jnp.max(s, axis=2, keepdims=True)
    e = jnp.exp(s - m)
    z = jnp.sum(e, axis=2, keepdims=True)
    pa_til = jnp.sum(e / z, axis=1)
    pa = jnp.dot(pa_til, perm_ref[...],
                 preferred_element_type=f32)

    iota = jax.lax.broadcasted_iota(jnp.int32, (_BB, 225), 1)
    vals = pa
    idx_list = []
    for _ in range(_FB):
        mv = jnp.max(vals, axis=1, keepdims=True)
        cand = jnp.where(vals >= mv, iota, 225)
        idx = jnp.min(cand, axis=1)
        idx_list.append(idx)
        vals = jnp.where(iota == idx[:, None], -1.0, vals)
    indices = jnp.stack(idx_list, axis=1)

    iota3 = jax.lax.broadcasted_iota(jnp.int32, (_BB, _FB, 225), 2)
    onehot = (iota3 == indices[:, :, None]).astype(f32)
    hsel = jax.lax.dot_general(
        onehot, hpre, (((2,), (1,)), ((0,), (0,))),
        preferred_element_type=f32)
    h = jnp.maximum(hsel, 0.0)
    ext = jax.lax.dot_general(
        h, wfe2t_ref[...], (((2,), (0,)), ((), ())),
        preferred_element_type=f32) + bfe2_ref[0]
    ext_flat = jnp.concatenate(
        [ext[:, t, :] for t in range(_FB)], axis=1)

    row = indices // 25
    col = indices % 25
    pos = jnp.concatenate([row * 4 + 4, col * 4 + 4], axis=1).astype(f32)
    features = jnp.concatenate([pos * (1.0 / 64.0), ext_flat], axis=1)

    o = jax.nn.sigmoid(
        jnp.dot(features, wc0t_ref[...], preferred_element_type=f32)
        + bc0_ref[0])
    o = jax.nn.sigmoid(
        jnp.dot(o, wc1t_ref[...], preferred_element_type=f32) + bc1_ref[0])
    logits = jnp.dot(o, wc2t_ref[...], preferred_element_type=f32) \
        + bc2_ref[0]

    liota = jax.lax.broadcasted_iota(jnp.int32, (_BB, 15), 1)
    lm = jnp.max(logits, axis=1, keepdims=True)
    action = jnp.min(jnp.where(logits >= lm, liota, 15), axis=1)
    out_ref[0, 0, :] = action


@jax.jit
def kernel(obs, Wq, bq, Wk, bk, Wfe1, bfe1, Wfe2, bfe2, Wc0, bc0, Wc1, bc1,
           Wc2, bc2):
    f32 = jnp.float32
    w11 = jnp.concatenate([Wq, Wk, Wfe1], axis=0)
    wtap = jnp.transpose(w11.reshape(11, 3, 49), (2, 1, 0))
    sel = jnp.asarray(_SEL_NP)
    perm = jnp.asarray(_PERM_NP)
    ceye = jnp.asarray(np.eye(15, dtype=np.float32))
    bd = (jnp.transpose(wtap, (0, 2, 1))[:, None, :, :, None]
          * ceye[None, :, None, None, :])
    bd = jnp.transpose(bd, (0, 1, 3, 2, 4)).reshape(49, 45, 165)
    b11 = jnp.concatenate([bq, bk, bfe1], axis=0)
    b165 = jnp.repeat(b11, 15).reshape(1, 165)

    nb = _NUM // _BB
    full = lambda shape: pl.BlockSpec(shape, lambda i: (0,) * len(shape))
    out = pl.pallas_call(
        _agent_block_kernel,
        grid=(nb,),
        in_specs=[
            pl.BlockSpec((_BB, 16, 4, 192), lambda i: (i, 0, 0, 0)),
            full((7, 192, 45)),
            full((49, 45, 165)),
            full((1, 165)),
            full((225, 225)),
            full((5, 3)),
            full((1, 3)),
            full((40, 20)),
            full((1, 20)),
            full((20, 15)),
            full((1, 15)),
            full((15, 15)),
            full((1, 15)),
        ],
        out_specs=pl.BlockSpec((1, 1, _BB), lambda i: (i, 0, 0)),
        out_shape=jax.ShapeDtypeStruct((nb, 1, _BB), jnp.int32),
    )(obs.reshape(_NUM, 16, 4, 192), sel, bd, b165, perm,
      jnp.transpose(Wfe2), bfe2.reshape(1, 3),
      jnp.transpose(Wc0), bc0.reshape(1, 20), jnp.transpose(Wc1),
      bc1.reshape(1, 15), jnp.transpose(Wc2), bc2.reshape(1, 15))
    return out.reshape(_NUM)


# speedup vs baseline: 1.1873x; 1.1873x over previous
import jax
import jax.numpy as jnp
import numpy as np
from jax.experimental import pallas as pl

_NUM = 1024
_FB = 8
_BB = 8

_PERM_NP = np.zeros((225, 225), np.float32)
for _m in range(75):
    for _c in range(3):
        _PERM_NP[75 * _c + _m, 3 * _m + _c] = 1.0

_SEL_NP = np.zeros((7, 192, 45), np.float32)
for _dx in range(7):
    for _c in range(15):
        for _ch in range(3):
            _SEL_NP[_dx, (4 * _c + _dx) * 3 + _ch, 3 * _c + _ch] = 1.0


def _agent_block_kernel(obs_ref, sel7_ref, bdd_ref, b165_ref, perm_ref,
                        wfe2t_ref, bfe2_ref, wc0t_ref, bc0_ref, wc1t_ref,
                        bc1_ref, wc2t_ref, bc2_ref, out_ref):
    f32 = jnp.float32
    x = obs_ref[...] * (1.0 / 255.0)

    ym = jnp.zeros((_BB * 15, 165), f32)
    sel7 = sel7_ref[...]
    for dy in range(7):
        r_dy = x[:, dy // 4:dy // 4 + 15, dy % 4, :]
        rm = r_dy.reshape(_BB * 15, 192)
        c_dy = jnp.dot(rm, sel7, preferred_element_type=f32)
        ym = ym + jnp.dot(c_dy, bdd_ref[dy],
                          preferred_element_type=f32)
    ym = ym + b165_ref[0]
    y = jnp.concatenate(
        [ym[None, b * 15:b * 15 + 15, :] for b in range(_BB)],
        axis=0)

    def out_mat(lo, hi):
        cols = [y[:, :, o * 15:o * 15 + 15][:, :, :, None]
                for o in range(lo, hi)]
        return jnp.concatenate(cols, axis=3).reshape(_BB, 225, hi - lo)

    q = out_mat(0, 3)
    k = out_mat(3, 6)
    hpre = out_mat(6, 11)

    krows = []
    for a in range(3):
        ka = k[:, 75 * a:75 * a + 75, :]
        krows.append(jnp.concatenate(
            [ka[:, :, 0], ka[:, :, 1], ka[:, :, 2]], axis=1)[:, None, :])
    ktil = jnp.concatenate(krows, axis=1)
    s = jax.lax.dot_general(
        q, ktil, (((2,), (1,)), ((0,), (0,))),
        preferred_element_type=f32) * (1.0 / np.sqrt(147.0))

    m = jnp.max(s, axis=2, keepdims=True)
    e = jnp.exp(s - m)
    z = jnp.sum(e, axis=2, keepdims=True)
    zr = (1.0 / z).reshape(_BB, 1, 225)
    pa_til = jax.lax.dot_general(
        zr, e, (((2,), (1,)), ((0,), (0,))),
        preferred_element_type=f32)[:, 0, :]
    pa = jnp.dot(pa_til, perm_ref[...],
                 preferred_element_type=f32)

    iota = jax.lax.broadcasted_iota(jnp.int32, (_BB, 225), 1)
    vals = pa
    idx_list = []
    for _ in range(_FB):
        mv = jnp.max(vals, axis=1, keepdims=True)
        cand = jnp.where(vals >= mv, iota, 225)
        idx = jnp.min(cand, axis=1)
        idx_list.append(idx)
        vals = jnp.where(iota == idx[:, None], -1.0, vals)
    indices = jnp.stack(idx_list, axis=1)

    iota3 = jax.lax.broadcasted_iota(jnp.int32, (_BB, _FB, 225), 2)
    onehot = (iota3 == indices[:, :, None]).astype(f32)
    hsel = jax.lax.dot_general(
        onehot, hpre, (((2,), (1,)), ((0,), (0,))),
        preferred_element_type=f32)
    h = jnp.maximum(hsel, 0.0)
    ext = jax.lax.dot_general(
        h, wfe2t_ref[...], (((2,), (0,)), ((), ())),
        preferred_element_type=f32) + bfe2_ref[0]
    ext_flat = jnp.concatenate(
        [ext[:, t, :] for t in range(_FB)], axis=1)

    row = indices // 25
    col = indices % 25
    pos = jnp.concatenate([row * 4 + 4, col * 4 + 4], axis=1).astype(f32)
    features = jnp.concatenate([pos * (1.0 / 64.0), ext_flat], axis=1)

    o = jax.nn.sigmoid(
        jnp.dot(features, wc0t_ref[...], preferred_element_type=f32)
        + bc0_ref[0])
    o = jax.nn.sigmoid(
        jnp.dot(o, wc1t_ref[...], preferred_element_type=f32) + bc1_ref[0])
    logits = jnp.dot(o, wc2t_ref[...], preferred_element_type=f32) \
        + bc2_ref[0]

    liota = jax.lax.broadcasted_iota(jnp.int32, (_BB, 15), 1)
    lm = jnp.max(logits, axis=1, keepdims=True)
    action = jnp.min(jnp.where(logits >= lm, liota, 15), axis=1)
    out_ref[0, 0, :] = action


@jax.jit
def kernel(obs, Wq, bq, Wk, bk, Wfe1, bfe1, Wfe2, bfe2, Wc0, bc0, Wc1, bc1,
           Wc2, bc2):
    f32 = jnp.float32
    w11 = jnp.concatenate([Wq, Wk, Wfe1], axis=0)
    wtap = jnp.transpose(w11.reshape(11, 3, 49), (2, 1, 0))
    sel7 = jnp.asarray(_SEL_NP).transpose(1, 0, 2).reshape(192, 7 * 45)
    perm = jnp.asarray(_PERM_NP)
    ceye = jnp.asarray(np.eye(15, dtype=np.float32))
    bd = (jnp.transpose(wtap, (0, 2, 1))[:, None, :, :, None]
          * ceye[None, :, None, None, :])
    bd = jnp.transpose(bd, (0, 1, 3, 2, 4)).reshape(49, 45, 165)
    bdd = bd.reshape(7, 7 * 45, 165)
    b11 = jnp.concatenate([bq, bk, bfe1], axis=0)
    b165 = jnp.repeat(b11, 15).reshape(1, 165)

    nb = _NUM // _BB
    full = lambda shape: pl.BlockSpec(shape, lambda i: (0,) * len(shape))
    out = pl.pallas_call(
        _agent_block_kernel,
        grid=(nb,),
        in_specs=[
            pl.BlockSpec((_BB, 16, 4, 192), lambda i: (i, 0, 0, 0)),
            full((192, 7 * 45)),
            full((7, 7 * 45, 165)),
            full((1, 165)),
            full((225, 225)),
            full((5, 3)),
            full((1, 3)),
            full((40, 20)),
            full((1, 20)),
            full((20, 15)),
            full((1, 15)),
            full((15, 15)),
            full((1, 15)),
        ],
        out_specs=pl.BlockSpec((1, 1, _BB), lambda i: (i, 0, 0)),
        out_shape=jax.ShapeDtypeStruct((nb, 1, _BB), jnp.int32),
    )(obs.reshape(_NUM, 16, 4, 192), sel7, bdd, b165, perm,
      jnp.transpose(Wfe2), bfe2.reshape(1, 3),
      jnp.transpose(Wc0), bc0.reshape(1, 20), jnp.transpose(Wc1),
      bc1.reshape(1, 15), jnp.transpose(Wc2), bc2.reshape(1, 15))
    return out.reshape(_NUM)


# speedup vs baseline: 1.8840x; 1.5867x over previous
import jax
import jax.numpy as jnp
import numpy as np
from jax.experimental import pallas as pl

_NUM = 1024
_FB = 8
_BB = 16

_PERM_NP = np.zeros((225, 225), np.float32)
for _m in range(75):
    for _c in range(3):
        _PERM_NP[75 * _c + _m, 3 * _m + _c] = 1.0

_SEL_NP = np.zeros((192, 7 * 45), np.float32)
for _dx in range(7):
    for _c in range(15):
        for _ch in range(3):
            _SEL_NP[(4 * _c + _dx) * 3 + _ch, _dx * 45 + 3 * _c + _ch] = 1.0


def _agent_kernel(obs_ref, sel7_ref, bdd_ref, b165_ref, perm_ref, wfe2t_ref,
                  bfe2_ref, wc0t_ref, bc0_ref, wc1t_ref, bc1_ref, wc2t_ref,
                  bc2_ref, out_ref):
    f32 = jnp.float32
    x = obs_ref[...] * (1.0 / 255.0)
    ym = jnp.zeros((_BB * 15, 165), f32)
    sel7 = sel7_ref[...]
    for dy in range(7):
        r_dy = x[:, dy // 4:dy // 4 + 15, dy % 4, :]
        rm = r_dy.reshape(_BB * 15, 192)
        c_dy = jnp.dot(rm, sel7, preferred_element_type=f32)
        ym = ym + jnp.dot(c_dy, bdd_ref[dy],
                          preferred_element_type=f32)
    ym = ym + b165_ref[0]

    y = jnp.transpose(ym.reshape(_BB, 15, 11, 15),
                      (0, 1, 3, 2)).reshape(_BB, 225, 11)

    q = y[:, :, 0:3] * (1.0 / np.sqrt(147.0))
    k = y[:, :, 3:6]
    hpre = y[:, :, 6:11]

    krows = []
    for a in range(3):
        ka = k[:, 75 * a:75 * a + 75, :]
        krows.append(jnp.concatenate(
            [ka[:, :, 0], ka[:, :, 1], ka[:, :, 2]], axis=1)[:, None, :])
    ktil = jnp.concatenate(krows, axis=1)
    s = jax.lax.dot_general(
        q, ktil, (((2,), (1,)), ((0,), (0,))),
        preferred_element_type=f32)

    e = jnp.exp(s)
    z = jnp.sum(e, axis=2, keepdims=True)
    zr = (1.0 / z).reshape(_BB, 1, 225)
    pa_til = jax.lax.dot_general(
        zr, e, (((2,), (1,)), ((0,), (0,))),
        preferred_element_type=f32)[:, 0, :]
    pa = jnp.dot(pa_til, perm_ref[...],
                 preferred_element_type=f32)

    iota = jax.lax.broadcasted_iota(jnp.int32, (_BB, 225), 1)
    vals = pa
    idx_list = []
    for _ in range(_FB):
        mv = jnp.max(vals, axis=1, keepdims=True)
        cand = jnp.where(vals >= mv, iota, 225)
        idx = jnp.min(cand, axis=1)
        idx_list.append(idx)
        vals = jnp.where(iota == idx[:, None], -1.0, vals)
    indices = jnp.stack(idx_list, axis=1)
    iota3 = jax.lax.broadcasted_iota(jnp.int32, (_BB, _FB, 225), 2)
    onehot = jnp.where(iota3 == indices[:, :, None], 1.0, 0.0)

    hsel = jax.lax.dot_general(
        onehot, hpre, (((2,), (1,)), ((0,), (0,))),
        preferred_element_type=f32)
    h = jnp.maximum(hsel, 0.0)
    ext = jax.lax.dot_general(
        h, wfe2t_ref[...], (((2,), (0,)), ((), ())),
        preferred_element_type=f32) + bfe2_ref[0]
    ext_flat = jnp.concatenate(
        [ext[:, t, :] for t in range(_FB)], axis=1)

    row = indices // 25
    col = indices % 25
    pos = jnp.concatenate([row * 4 + 4, col * 4 + 4], axis=1).astype(f32)
    features = jnp.concatenate([pos * (1.0 / 64.0), ext_flat], axis=1)

    o = jax.nn.sigmoid(
        jnp.dot(features, wc0t_ref[...], preferred_element_type=f32)
        + bc0_ref[0])
    o = jax.nn.sigmoid(
        jnp.dot(o, wc1t_ref[...], preferred_element_type=f32) + bc1_ref[0])
    logits = jnp.dot(o, wc2t_ref[...], preferred_element_type=f32) \
        + bc2_ref[0]

    liota = jax.lax.broadcasted_iota(jnp.int32, (_BB, 15), 1)
    lm = jnp.max(logits, axis=1, keepdims=True)
    action = jnp.min(jnp.where(logits >= lm, liota, 15), axis=1)
    out_ref[0, 0, :] = action


@jax.jit
def kernel(obs, Wq, bq, Wk, bk, Wfe1, bfe1, Wfe2, bfe2, Wc0, bc0, Wc1, bc1,
           Wc2, bc2):
    w11 = jnp.concatenate([Wq, Wk, Wfe1], axis=0)
    wtap = jnp.transpose(w11.reshape(11, 3, 49), (2, 1, 0))
    sel7 = jnp.asarray(_SEL_NP)
    perm = jnp.asarray(_PERM_NP)
    ceye = jnp.asarray(np.eye(15, dtype=np.float32))
    bd = (jnp.transpose(wtap, (0, 2, 1))[:, None, :, :, None]
          * ceye[None, :, None, None, :])
    bd = jnp.transpose(bd, (0, 1, 3, 2, 4)).reshape(49, 45, 165)
    bdd = bd.reshape(7, 7 * 45, 165)
    b11 = jnp.concatenate([bq, bk, bfe1], axis=0)
    b165 = jnp.repeat(b11, 15).reshape(1, 165)

    nb = _NUM // _BB
    full = lambda shape: pl.BlockSpec(shape, lambda i: (0,) * len(shape))
    out = pl.pallas_call(
        _agent_kernel,
        grid=(nb,),
        in_specs=[
            pl.BlockSpec((_BB, 16, 4, 192), lambda i: (i, 0, 0, 0)),
            full((192, 7 * 45)),
            full((7, 7 * 45, 165)),
            full((1, 165)),
            full((225, 225)),
            full((5, 3)),
            full((1, 3)),
            full((40, 20)),
            full((1, 20)),
            full((20, 15)),
            full((1, 15)),
            full((15, 15)),
            full((1, 15)),
        ],
        out_specs=pl.BlockSpec((1, 1, _BB), lambda i: (i, 0, 0)),
        out_shape=jax.ShapeDtypeStruct((nb, 1, _BB), jnp.int32),
    )(obs.reshape(_NUM, 16, 4, 192), sel7, bdd, b165, perm,
      jnp.transpose(Wfe2), bfe2.reshape(1, 3),
      jnp.transpose(Wc0), bc0.reshape(1, 20), jnp.transpose(Wc1),
      bc1.reshape(1, 15), jnp.transpose(Wc2), bc2.reshape(1, 15))
    return out.reshape(_NUM)


# speedup vs baseline: 1.9440x; 1.0319x over previous
import jax
import jax.numpy as jnp
import numpy as np
from jax.experimental import pallas as pl

_NUM = 1024
_FB = 8
_BB = 16

_PERM_NP = np.zeros((225, 225), np.float32)
for _m in range(75):
    for _c in range(3):
        _PERM_NP[75 * _c + _m, 3 * _m + _c] = 1.0

_SEL_NP = np.zeros((192, 7 * 45), np.float32)
for _dx in range(7):
    for _c in range(15):
        for _ch in range(3):
            _SEL_NP[(4 * _c + _dx) * 3 + _ch, _dx * 45 + 3 * _c + _ch] = 1.0


def _agent_kernel(obs_ref, sel7_ref, bdd_ref, b165_ref, perm_ref, wfe2t_ref,
                  bfe2_ref, wc0t_ref, bc0_ref, wc1t_ref, bc1_ref, wc2t_ref,
                  bc2_ref, out_ref):
    f32 = jnp.float32
    x = obs_ref[...] * (1.0 / 255.0)
    ym = jnp.zeros((_BB * 15, 165), f32)
    sel7 = sel7_ref[...]
    for dy in range(7):
        r_dy = x[:, dy // 4:dy // 4 + 15,
                 (dy % 4) * 192:(dy % 4) * 192 + 192]
        rm = r_dy.reshape(_BB * 15, 192)
        c_dy = jnp.dot(rm, sel7, preferred_element_type=f32)
        ym = ym + jnp.dot(c_dy, bdd_ref[dy],
                          preferred_element_type=f32)
    ym = ym + b165_ref[0]

    y = jnp.transpose(ym.reshape(_BB, 15, 11, 15),
                      (0, 1, 3, 2)).reshape(_BB, 225, 11)

    q = y[:, :, 0:3] * (1.0 / np.sqrt(147.0))
    k = y[:, :, 3:6]
    hpre = y[:, :, 6:11]

    krows = []
    for a in range(3):
        ka = k[:, 75 * a:75 * a + 75, :]
        krows.append(jnp.concatenate(
            [ka[:, :, 0], ka[:, :, 1], ka[:, :, 2]], axis=1)[:, None, :])
    ktil = jnp.concatenate(krows, axis=1)
    s = jax.lax.dot_general(
        q, ktil, (((2,), (1,)), ((0,), (0,))),
        preferred_element_type=f32)

    e = jnp.exp(s)
    z = jnp.sum(e, axis=2, keepdims=True)
    zr = (1.0 / z).reshape(_BB, 1, 225)
    pa_til = jax.lax.dot_general(
        zr, e, (((2,), (1,)), ((0,), (0,))),
        preferred_element_type=f32)[:, 0, :]
    pa = jnp.dot(pa_til, perm_ref[...],
                 preferred_element_type=f32)

    iota = jax.lax.broadcasted_iota(jnp.int32, (_BB, 225), 1)
    vals = pa
    idx_list = []
    for _ in range(_FB):
        mv = jnp.max(vals, axis=1, keepdims=True)
        cand = jnp.where(vals >= mv, iota, 225)
        idx = jnp.min(cand, axis=1)
        idx_list.append(idx)
        vals = jnp.where(iota == idx[:, None], -1.0, vals)
    indices = jnp.stack(idx_list, axis=1)
    iota3 = jax.lax.broadcasted_iota(jnp.int32, (_BB, _FB, 225), 2)
    onehot = jnp.where(iota3 == indices[:, :, None], 1.0, 0.0)

    hsel = jax.lax.dot_general(
        onehot, hpre, (((2,), (1,)), ((0,), (0,))),
        preferred_element_type=f32)
    h = jnp.maximum(hsel, 0.0)
    ext = jax.lax.dot_general(
        h, wfe2t_ref[...], (((2,), (0,)), ((), ())),
        preferred_element_type=f32) + bfe2_ref[0]
    ext_flat = jnp.concatenate(
        [ext[:, t, :] for t in range(_FB)], axis=1)

    row = indices // 25
    col = indices % 25
    pos = jnp.concatenate([row * 4 + 4, col * 4 + 4], axis=1).astype(f32)
    features = jnp.concatenate([pos * (1.0 / 64.0), ext_flat], axis=1)

    o = jax.nn.sigmoid(
        jnp.dot(features, wc0t_ref[...], preferred_element_type=f32)
        + bc0_ref[0])
    o = jax.nn.sigmoid(
        jnp.dot(o, wc1t_ref[...], preferred_element_type=f32) + bc1_ref[0])
    logits = jnp.dot(o, wc2t_ref[...], preferred_element_type=f32) \
        + bc2_ref[0]

    liota = jax.lax.broadcasted_iota(jnp.int32, (_BB, 15), 1)
    lm = jnp.max(logits, axis=1, keepdims=True)
    action = jnp.min(jnp.where(logits >= lm, liota, 15), axis=1)
    out_ref[0, 0, :] = action


@jax.jit
def kernel(obs, Wq, bq, Wk, bk, Wfe1, bfe1, Wfe2, bfe2, Wc0, bc0, Wc1, bc1,
           Wc2, bc2):
    w11 = jnp.concatenate([Wq, Wk, Wfe1], axis=0)
    wtap = jnp.transpose(w11.reshape(11, 3, 49), (2, 1, 0))
    sel7 = jnp.asarray(_SEL_NP)
    perm = jnp.asarray(_PERM_NP)
    ceye = jnp.asarray(np.eye(15, dtype=np.float32))
    bd = (jnp.transpose(wtap, (0, 2, 1))[:, None, :, :, None]
          * ceye[None, :, None, None, :])
    bd = jnp.transpose(bd, (0, 1, 3, 2, 4)).reshape(49, 45, 165)
    bdd = bd.reshape(7, 7 * 45, 165)
    b11 = jnp.concatenate([bq, bk, bfe1], axis=0)
    b165 = jnp.repeat(b11, 15).reshape(1, 165)

    nb = _NUM // _BB
    full = lambda shape: pl.BlockSpec(shape, lambda i: (0,) * len(shape))
    out = pl.pallas_call(
        _agent_kernel,
        grid=(nb,),
        in_specs=[
            pl.BlockSpec((_BB, 16, 768), lambda i: (i, 0, 0)),
            full((192, 7 * 45)),
            full((7, 7 * 45, 165)),
            full((1, 165)),
            full((225, 225)),
            full((5, 3)),
            full((1, 3)),
            full((40, 20)),
            full((1, 20)),
            full((20, 15)),
            full((1, 15)),
            full((15, 15)),
            full((1, 15)),
        ],
        out_specs=pl.BlockSpec((1, 1, _BB), lambda i: (i, 0, 0)),
        out_shape=jax.ShapeDtypeStruct((nb, 1, _BB), jnp.int32),
    )(obs.reshape(_NUM, 16, 768), sel7, bdd, b165, perm,
      jnp.transpose(Wfe2), bfe2.reshape(1, 3),
      jnp.transpose(Wc0), bc0.reshape(1, 20), jnp.transpose(Wc1),
      bc1.reshape(1, 15), jnp.transpose(Wc2), bc2.reshape(1, 15))
    return out.reshape(_NUM)


# speedup vs baseline: 2.2377x; 1.1510x over previous
import jax
import jax.numpy as jnp
import numpy as np
from jax.experimental import pallas as pl

_NUM = 1024
_FB = 8
_BB = 32

_PERM_NP = np.zeros((225, 225), np.float32)
for _m in range(75):
    for _c in range(3):
        _PERM_NP[75 * _c + _m, 3 * _m + _c] = 1.0

_SEL_NP = np.zeros((192, 7 * 45), np.float32)
for _dx in range(7):
    for _c in range(15):
        for _ch in range(3):
            _SEL_NP[(4 * _c + _dx) * 3 + _ch, _dx * 45 + 3 * _c + _ch] = 1.0


def _agent_kernel(obs_ref, sel7_ref, bdd_ref, b165_ref, perm_ref, wfe2t_ref,
                  bfe2_ref, wc0t_ref, bc0_ref, wc1t_ref, bc1_ref, wc2t_ref,
                  bc2_ref, out_ref):
    f32 = jnp.float32
    x = obs_ref[...] * (1.0 / 255.0)
    ym = jnp.zeros((_BB * 15, 165), f32)
    sel7 = sel7_ref[...]
    for dy in range(7):
        r_dy = x[:, dy // 4:dy // 4 + 15,
                 (dy % 4) * 192:(dy % 4) * 192 + 192]
        rm = r_dy.reshape(_BB * 15, 192)
        c_dy = jnp.dot(rm, sel7, preferred_element_type=f32)
        ym = ym + jnp.dot(c_dy, bdd_ref[dy],
                          preferred_element_type=f32)
    ym = ym + b165_ref[0]

    y = jnp.transpose(ym.reshape(_BB, 15, 11, 15),
                      (0, 1, 3, 2)).reshape(_BB, 225, 11)

    q = y[:, :, 0:3] * (1.0 / np.sqrt(147.0))
    k = y[:, :, 3:6]
    hpre = y[:, :, 6:11]

    krows = []
    for a in range(3):
        ka = k[:, 75 * a:75 * a + 75, :]
        krows.append(jnp.concatenate(
            [ka[:, :, 0], ka[:, :, 1], ka[:, :, 2]], axis=1)[:, None, :])
    ktil = jnp.concatenate(krows, axis=1)
    s = jax.lax.dot_general(
        q, ktil, (((2,), (1,)), ((0,), (0,))),
        preferred_element_type=f32)

    e = jnp.exp(s)
    z = jnp.sum(e, axis=2, keepdims=True)
    zr = (1.0 / z).reshape(_BB, 1, 225)
    pa_til = jax.lax.dot_general(
        zr, e, (((2,), (1,)), ((0,), (0,))),
        preferred_element_type=f32)[:, 0, :]
    pa = jnp.dot(pa_til, perm_ref[...],
                 preferred_element_type=f32)

    iota = jax.lax.broadcasted_iota(jnp.int32, (_BB, 225), 1)
    vals = pa
    idx_list = []
    for _ in range(_FB):
        mv = jnp.max(vals, axis=1, keepdims=True)
        cand = jnp.where(vals >= mv, iota, 225)
        idx = jnp.min(cand, axis=1)
        idx_list.append(idx)
        vals = jnp.where(iota == idx[:, None], -1.0, vals)
    indices = jnp.stack(idx_list, axis=1)
    iota3 = jax.lax.broadcasted_iota(jnp.int32, (_BB, _FB, 225), 2)
    onehot = jnp.where(iota3 == indices[:, :, None], 1.0, 0.0)

    hsel = jax.lax.dot_general(
        onehot, hpre, (((2,), (1,)), ((0,), (0,))),
        preferred_element_type=f32)
    h = jnp.maximum(hsel, 0.0)
    ext = jax.lax.dot_general(
        h, wfe2t_ref[...], (((2,), (0,)), ((), ())),
        preferred_element_type=f32) + bfe2_ref[0]
    ext_flat = jnp.concatenate(
        [ext[:, t, :] for t in range(_FB)], axis=1)

    row = indices // 25
    col = indices % 25
    pos = jnp.concatenate([row * 4 + 4, col * 4 + 4], axis=1).astype(f32)
    features = jnp.concatenate([pos * (1.0 / 64.0), ext_flat], axis=1)

    o = jax.nn.sigmoid(
        jnp.dot(features, wc0t_ref[...], preferred_element_type=f32)
        + bc0_ref[0])
    o = jax.nn.sigmoid(
        jnp.dot(o, wc1t_ref[...], preferred_element_type=f32) + bc1_ref[0])
    logits = jnp.dot(o, wc2t_ref[...], preferred_element_type=f32) \
        + bc2_ref[0]

    liota = jax.lax.broadcasted_iota(jnp.int32, (_BB, 15), 1)
    lm = jnp.max(logits, axis=1, keepdims=True)
    action = jnp.min(jnp.where(logits >= lm, liota, 15), axis=1)
    out_ref[0, 0, :] = action


@jax.jit
def kernel(obs, Wq, bq, Wk, bk, Wfe1, bfe1, Wfe2, bfe2, Wc0, bc0, Wc1, bc1,
           Wc2, bc2):
    w11 = jnp.concatenate([Wq, Wk, Wfe1], axis=0)
    wtap = jnp.transpose(w11.reshape(11, 3, 49), (2, 1, 0))
    sel7 = jnp.asarray(_SEL_NP)
    perm = jnp.asarray(_PERM_NP)
    ceye = jnp.asarray(np.eye(15, dtype=np.float32))
    bd = (jnp.transpose(wtap, (0, 2, 1))[:, None, :, :, None]
          * ceye[None, :, None, None, :])
    bd = jnp.transpose(bd, (0, 1, 3, 2, 4)).reshape(49, 45, 165)
    bdd = bd.reshape(7, 7 * 45, 165)
    b11 = jnp.concatenate([bq, bk, bfe1], axis=0)
    b165 = jnp.repeat(b11, 15).reshape(1, 165)

    nb = _NUM // _BB
    full = lambda shape: pl.BlockSpec(shape, lambda i: (0,) * len(shape))
    out = pl.pallas_call(
        _agent_kernel,
        grid=(nb,),
        in_specs=[
            pl.BlockSpec((_BB, 16, 768), lambda i: (i, 0, 0)),
            full((192, 7 * 45)),
            full((7, 7 * 45, 165)),
            full((1, 165)),
            full((225, 225)),
            full((5, 3)),
            full((1, 3)),
            full((40, 20)),
            full((1, 20)),
            full((20, 15)),
            full((1, 15)),
            full((15, 15)),
            full((1, 15)),
        ],
        out_specs=pl.BlockSpec((1, 1, _BB), lambda i: (i, 0, 0)),
        out_shape=jax.ShapeDtypeStruct((nb, 1, _BB), jnp.int32),
    )(obs.reshape(_NUM, 16, 768), sel7, bdd, b165, perm,
      jnp.transpose(Wfe2), bfe2.reshape(1, 3),
      jnp.transpose(Wc0), bc0.reshape(1, 20), jnp.transpose(Wc1),
      bc1.reshape(1, 15), jnp.transpose(Wc2), bc2.reshape(1, 15))
    return out.reshape(_NUM)
